# baseline (speedup 1.0000x reference)
import functools

import jax
import jax.numpy as jnp
from jax import lax
from jax.experimental import pallas as pl
from jax.experimental.pallas import tpu as pltpu
from jax.experimental.pallas import tpu_sc as plsc

_NC = 2
_NS = 16
_LANES = 128
_F0 = 0.25


@functools.lru_cache(maxsize=None)
def _make_deg_pass(nch, n_pad_deg):
  mesh = plsc.VectorSubcoreMesh(core_axis_name="c", subcore_axis_name="s")
  dpt = n_pad_deg // _NS

  def body(dst_hbm, deg_out, dst_v, ones_v, dzero, deg_sh):
    c = lax.axis_index("c")
    s = lax.axis_index("s")
    w = s * _NC + c

    for k in range(_LANES // 16):
      ones_v[pl.ds(16 * k, 16)] = jnp.ones((16,), jnp.float32)
    for k in range(dpt // 16):
      dzero[pl.ds(16 * k, 16)] = jnp.zeros((16,), jnp.float32)
    pltpu.sync_copy(dzero, deg_sh.at[pl.ds(s * dpt, dpt)])
    plsc.subcore_barrier()

    pltpu.sync_copy(dst_hbm.at[pl.ds(pl.multiple_of(w * nch, 8), nch)], dst_v)

    def step(j, carry):
      pltpu.sync_copy(ones_v, deg_sh.at[dst_v.at[j]], add=True)
      return carry
    lax.fori_loop(0, nch, step, 0)

    plsc.subcore_barrier()
    pltpu.sync_copy(deg_sh.at[pl.ds(s * dpt, dpt)],
                    deg_out.at[pl.ds(c * n_pad_deg + s * dpt, dpt)])

  return pl.kernel(
      body,
      out_type=jax.ShapeDtypeStruct((_NC * n_pad_deg,), jnp.float32),
      mesh=mesh,
      scratch_types=[
          pltpu.VMEM((nch, _LANES), jnp.int32),
          pltpu.VMEM((_LANES,), jnp.float32),
          pltpu.VMEM((dpt,), jnp.float32),
          pltpu.VMEM_SHARED((n_pad_deg,), jnp.float32),
      ],
  )


@functools.lru_cache(maxsize=None)
def _make_sc_pass(nch0, nch1, n_pad):
  mesh = plsc.VectorSubcoreMesh(core_axis_name="c", subcore_axis_name="s")
  rpt = n_pad // _NS
  grp = 8
  tot = nch0 + nch1
  nmax = max(nch0, nch1)

  def body(x_hbm, src_hbm, dst_hbm, part_out, src_v, dst_g, rows0, rows1,
           acc_sh, sem0, sem1):
    c = lax.axis_index("c")
    s = lax.axis_index("s")
    base = pl.multiple_of(s * tot + c * nch0, 8)

    def zrow(i, carry):
      for k in range(128 // 16):
        rows0[i, pl.ds(16 * k, 16)] = jnp.zeros((16,), jnp.float32)
      return carry
    lax.fori_loop(0, _LANES, zrow, 0)

    off = s * rpt
    nfull = rpt // _LANES
    rem = rpt - nfull * _LANES
    for t in range(nfull):
      pltpu.sync_copy(rows0, acc_sh.at[pl.ds(off + t * _LANES, _LANES)])
    if rem:
      pltpu.sync_copy(rows0.at[pl.ds(0, rem)],
                      acc_sh.at[pl.ds(off + nfull * _LANES, rem)])

    plsc.subcore_barrier()

    pltpu.sync_copy(src_hbm.at[pl.ds(base, nmax)], src_v)
    pltpu.sync_copy(dst_hbm.at[pl.ds(base, grp)], dst_g)
    pltpu.async_copy(x_hbm.at[src_v.at[0]], rows0, sem0)
    pltpu.async_copy(x_hbm.at[src_v.at[1]], rows1, sem1)

    def run(cnt):
      def pair(jj, carry):
        j0 = 2 * jj
        j1 = j0 + 1

        @pl.when(jnp.logical_and(lax.rem(j0, grp) == 0, jj != 0))
        def _():
          g = lax.div(j0, grp)
          pltpu.sync_copy(
              dst_hbm.at[pl.ds(pl.multiple_of(base + g * grp, 8), grp)],
              dst_g)

        r0 = lax.rem(j0, grp)
        for rows, sem, j, r in ((rows0, sem0, j0, r0),
                                (rows1, sem1, j1, r0 + 1)):
          pltpu.make_async_copy(x_hbm.at[src_v.at[j]], rows, sem).wait()
          pltpu.sync_copy(rows, acc_sh.at[dst_g.at[r]], add=True)
          jn = jnp.minimum(j + 2, cnt - 1)
          pltpu.async_copy(x_hbm.at[src_v.at[jn]], rows, sem)
        return carry
      lax.fori_loop(0, cnt // 2, pair, 0)

      pltpu.make_async_copy(x_hbm.at[src_v.at[cnt - 1]], rows0, sem0).wait()
      pltpu.make_async_copy(x_hbm.at[src_v.at[cnt - 1]], rows1, sem1).wait()

    @pl.when(c == 0)
    def _():
      run(nch0)

    @pl.when(c == 1)
    def _():
      run(nch1)

    plsc.subcore_barrier()

    pltpu.sync_copy(acc_sh.at[pl.ds(s * rpt, rpt)],
                    part_out.at[c, pl.ds(s * rpt, rpt)])

  return pl.kernel(
      body,
      out_type=jax.ShapeDtypeStruct((_NC, n_pad, 128), jnp.float32),
      mesh=mesh,
      scratch_types=[
          pltpu.VMEM((nmax, _LANES), jnp.int32),
          pltpu.VMEM((grp, _LANES), jnp.int32),
          pltpu.VMEM((_LANES, 128), jnp.float32),
          pltpu.VMEM((_LANES, 128), jnp.float32),
          pltpu.VMEM_SHARED((n_pad, 128), jnp.float32),
          pltpu.SemaphoreType.DMA,
          pltpu.SemaphoreType.DMA,
      ],
  )


@functools.lru_cache(maxsize=None)
def _make_tc_stage(n, blk, residual):
  grid = (n // blk,)

  def body(part_ref, degp_ref, w_ref, b_ref, *rest):
    if residual:
      x_ref, out_ref = rest
    else:
      (out_ref,) = rest
    p = part_ref[0] + part_ref[1]
    deg = degp_ref[0] + degp_ref[1]
    dinv = 1.0 / jnp.maximum(deg, 1.0)
    h = jnp.dot(p * dinv, w_ref[...],
                preferred_element_type=jnp.float32) + b_ref[...]
    if residual:
      h = h + x_ref[...]
    out_ref[...] = jnp.maximum(h, 0.0)

  in_specs = [
      pl.BlockSpec((_NC, blk, 128), lambda i: (0, i, 0)),
      pl.BlockSpec((_NC, blk, 1), lambda i: (0, i, 0)),
      pl.BlockSpec((128, 128), lambda i: (0, 0)),
      pl.BlockSpec((1, 128), lambda i: (0, 0)),
  ]
  if residual:
    in_specs.append(pl.BlockSpec((blk, 128), lambda i: (i, 0)))

  return pl.pallas_call(
      body,
      grid=grid,
      in_specs=in_specs,
      out_specs=pl.BlockSpec((blk, 128), lambda i: (i, 0)),
      out_shape=jax.ShapeDtypeStruct((n, 128), jnp.float32),
  )


def kernel(x, edge_index, W1_0, b1_0, W2_0, b2_0, W1_1, b1_1, W2_1, b2_1):
  n, d = x.shape
  e = edge_index.shape[1]
  assert d == 128

  workers = _NC * _NS
  tot = 8 * (-(-e // (_NS * _LANES * 8)))
  nch0 = 8 * int(round(tot * _F0 / 8))
  nch0 = min(max(nch0, 8), tot - 8)
  nch1 = tot - nch0
  rows_needed = (_NS - 1) * tot + nch0 + max(nch0, nch1)
  e_pad = max(_NS * tot, rows_needed) * _LANES
  n_pad = 128 * (-(-(n + 1) // 128))
  n_pad_deg = _NS * (16 * (-(-(n_pad // _NS) // 16)))
  nch_deg = tot // 2

  pad = e_pad - e
  src = jnp.concatenate(
      [edge_index[0], jnp.zeros((pad,), jnp.int32)]).reshape(-1, _LANES)
  dst = jnp.concatenate(
      [edge_index[1], jnp.full((pad,), n, jnp.int32)]).reshape(-1, _LANES)

  sc_deg = _make_deg_pass(nch_deg, n_pad_deg)
  sc_seg = _make_sc_pass(nch0, nch1, n_pad)

  blk = 1000 if n % 1000 == 0 else 8 * (n // 8)
  tc_mid = _make_tc_stage(n, blk, False)
  tc_res = _make_tc_stage(n, blk, True)

  degp = sc_deg(dst)
  degp3 = degp.reshape(_NC, n_pad_deg, 1)

  part1 = sc_seg(x, src, dst)
  h = tc_mid(part1, degp3, W1_0, b1_0.reshape(1, 128))
  part2 = sc_seg(h, src, dst)
  x1 = tc_res(part2, degp3, W2_0, b2_0.reshape(1, 128), x)

  part3 = sc_seg(x1, src, dst)
  h2 = tc_mid(part3, degp3, W1_1, b1_1.reshape(1, 128))
  part4 = sc_seg(h2, src, dst)
  x2 = tc_res(part4, degp3, W2_1, b2_1.reshape(1, 128), x1)

  return (x2, edge_index)


# speedup vs baseline: 1.1019x; 1.1019x over previous
import functools

import jax
import jax.numpy as jnp
from jax import lax
from jax.experimental import pallas as pl
from jax.experimental.pallas import tpu as pltpu
from jax.experimental.pallas import tpu_sc as plsc

_NC = 2
_NS = 16
_LANES = 128
_F0 = 0.7


@functools.lru_cache(maxsize=None)
def _make_deg_pass(nch, n_pad_deg):
  mesh = plsc.VectorSubcoreMesh(core_axis_name="c", subcore_axis_name="s")
  dpt = n_pad_deg // _NS

  def body(dst_hbm, deg_out, dst_v, ones_v, dzero, deg_sh):
    c = lax.axis_index("c")
    s = lax.axis_index("s")
    w = s * _NC + c

    for k in range(_LANES // 16):
      ones_v[pl.ds(16 * k, 16)] = jnp.ones((16,), jnp.float32)
    for k in range(dpt // 16):
      dzero[pl.ds(16 * k, 16)] = jnp.zeros((16,), jnp.float32)
    pltpu.sync_copy(dzero, deg_sh.at[pl.ds(s * dpt, dpt)])
    plsc.subcore_barrier()

    pltpu.sync_copy(dst_hbm.at[pl.ds(pl.multiple_of(w * nch, 8), nch)], dst_v)

    def step(j, carry):
      pltpu.sync_copy(ones_v, deg_sh.at[dst_v.at[j]], add=True)
      return carry
    lax.fori_loop(0, nch, step, 0)

    plsc.subcore_barrier()
    pltpu.sync_copy(deg_sh.at[pl.ds(s * dpt, dpt)],
                    deg_out.at[pl.ds(c * n_pad_deg + s * dpt, dpt)])

  return pl.kernel(
      body,
      out_type=jax.ShapeDtypeStruct((_NC * n_pad_deg,), jnp.float32),
      mesh=mesh,
      scratch_types=[
          pltpu.VMEM((nch, _LANES), jnp.int32),
          pltpu.VMEM((_LANES,), jnp.float32),
          pltpu.VMEM((dpt,), jnp.float32),
          pltpu.VMEM_SHARED((n_pad_deg,), jnp.float32),
      ],
  )


@functools.lru_cache(maxsize=None)
def _make_sc_pass(nch0, nch1, n_pad):
  mesh = plsc.VectorSubcoreMesh(core_axis_name="c", subcore_axis_name="s")
  rpt = n_pad // _NS
  grp = 8
  tot = nch0 + nch1
  nmax = max(nch0, nch1)

  def body(x_hbm, src_hbm, dst_hbm, part_out, src_v, dst_g, rows0, rows1,
           acc_sh, sem0, sem1):
    c = lax.axis_index("c")
    s = lax.axis_index("s")
    base = pl.multiple_of(s * tot + c * nch0, 8)

    def zrow(i, carry):
      for k in range(128 // 16):
        rows0[i, pl.ds(16 * k, 16)] = jnp.zeros((16,), jnp.float32)
      return carry
    lax.fori_loop(0, _LANES, zrow, 0)

    off = s * rpt
    nfull = rpt // _LANES
    rem = rpt - nfull * _LANES
    for t in range(nfull):
      pltpu.sync_copy(rows0, acc_sh.at[pl.ds(off + t * _LANES, _LANES)])
    if rem:
      pltpu.sync_copy(rows0.at[pl.ds(0, rem)],
                      acc_sh.at[pl.ds(off + nfull * _LANES, rem)])

    plsc.subcore_barrier()

    pltpu.sync_copy(src_hbm.at[pl.ds(base, nmax)], src_v)
    pltpu.sync_copy(dst_hbm.at[pl.ds(base, grp)], dst_g)
    pltpu.async_copy(x_hbm.at[src_v.at[0]], rows0, sem0)
    pltpu.async_copy(x_hbm.at[src_v.at[1]], rows1, sem1)

    def run(cnt):
      def pair(jj, carry):
        j0 = 2 * jj
        j1 = j0 + 1

        @pl.when(jnp.logical_and(lax.rem(j0, grp) == 0, jj != 0))
        def _():
          g = lax.div(j0, grp)
          pltpu.sync_copy(
              dst_hbm.at[pl.ds(pl.multiple_of(base + g * grp, 8), grp)],
              dst_g)

        r0 = lax.rem(j0, grp)
        for rows, sem, j, r in ((rows0, sem0, j0, r0),
                                (rows1, sem1, j1, r0 + 1)):
          pltpu.make_async_copy(x_hbm.at[src_v.at[j]], rows, sem).wait()
          pltpu.sync_copy(rows, acc_sh.at[dst_g.at[r]], add=True)
          jn = jnp.minimum(j + 2, cnt - 1)
          pltpu.async_copy(x_hbm.at[src_v.at[jn]], rows, sem)
        return carry
      lax.fori_loop(0, cnt // 2, pair, 0)

      pltpu.make_async_copy(x_hbm.at[src_v.at[cnt - 1]], rows0, sem0).wait()
      pltpu.make_async_copy(x_hbm.at[src_v.at[cnt - 1]], rows1, sem1).wait()

    @pl.when(c == 0)
    def _():
      run(nch0)

    @pl.when(c == 1)
    def _():
      run(nch1)

    plsc.subcore_barrier()

    pltpu.sync_copy(acc_sh.at[pl.ds(s * rpt, rpt)],
                    part_out.at[c, pl.ds(s * rpt, rpt)])

  return pl.kernel(
      body,
      out_type=jax.ShapeDtypeStruct((_NC, n_pad, 128), jnp.float32),
      mesh=mesh,
      scratch_types=[
          pltpu.VMEM((nmax, _LANES), jnp.int32),
          pltpu.VMEM((grp, _LANES), jnp.int32),
          pltpu.VMEM((_LANES, 128), jnp.float32),
          pltpu.VMEM((_LANES, 128), jnp.float32),
          pltpu.VMEM_SHARED((n_pad, 128), jnp.float32),
          pltpu.SemaphoreType.DMA,
          pltpu.SemaphoreType.DMA,
      ],
  )


@functools.lru_cache(maxsize=None)
def _make_tc_stage(n, blk, residual):
  grid = (n // blk,)

  def body(part_ref, degp_ref, w_ref, b_ref, *rest):
    if residual:
      x_ref, out_ref = rest
    else:
      (out_ref,) = rest
    p = part_ref[0] + part_ref[1]
    deg = degp_ref[0] + degp_ref[1]
    dinv = 1.0 / jnp.maximum(deg, 1.0)
    h = jnp.dot(p * dinv, w_ref[...],
                preferred_element_type=jnp.float32) + b_ref[...]
    if residual:
      h = h + x_ref[...]
    out_ref[...] = jnp.maximum(h, 0.0)

  in_specs = [
      pl.BlockSpec((_NC, blk, 128), lambda i: (0, i, 0)),
      pl.BlockSpec((_NC, blk, 1), lambda i: (0, i, 0)),
      pl.BlockSpec((128, 128), lambda i: (0, 0)),
      pl.BlockSpec((1, 128), lambda i: (0, 0)),
  ]
  if residual:
    in_specs.append(pl.BlockSpec((blk, 128), lambda i: (i, 0)))

  return pl.pallas_call(
      body,
      grid=grid,
      in_specs=in_specs,
      out_specs=pl.BlockSpec((blk, 128), lambda i: (i, 0)),
      out_shape=jax.ShapeDtypeStruct((n, 128), jnp.float32),
  )


def kernel(x, edge_index, W1_0, b1_0, W2_0, b2_0, W1_1, b1_1, W2_1, b2_1):
  n, d = x.shape
  e = edge_index.shape[1]
  assert d == 128

  workers = _NC * _NS
  tot = 8 * (-(-e // (_NS * _LANES * 8)))
  nch0 = 8 * int(round(tot * _F0 / 8))
  nch0 = min(max(nch0, 8), tot - 8)
  nch1 = tot - nch0
  rows_needed = (_NS - 1) * tot + nch0 + max(nch0, nch1)
  e_pad = max(_NS * tot, rows_needed) * _LANES
  n_pad = 128 * (-(-(n + 1) // 128))
  n_pad_deg = _NS * (16 * (-(-(n_pad // _NS) // 16)))
  nch_deg = tot // 2

  pad = e_pad - e
  src = jnp.concatenate(
      [edge_index[0], jnp.zeros((pad,), jnp.int32)]).reshape(-1, _LANES)
  dst = jnp.concatenate(
      [edge_index[1], jnp.full((pad,), n, jnp.int32)]).reshape(-1, _LANES)

  sc_deg = _make_deg_pass(nch_deg, n_pad_deg)
  sc_seg = _make_sc_pass(nch0, nch1, n_pad)

  blk = 1000 if n % 1000 == 0 else 8 * (n // 8)
  tc_mid = _make_tc_stage(n, blk, False)
  tc_res = _make_tc_stage(n, blk, True)

  degp = sc_deg(dst)
  degp3 = degp.reshape(_NC, n_pad_deg, 1)

  part1 = sc_seg(x, src, dst)
  h = tc_mid(part1, degp3, W1_0, b1_0.reshape(1, 128))
  part2 = sc_seg(h, src, dst)
  x1 = tc_res(part2, degp3, W2_0, b2_0.reshape(1, 128), x)

  part3 = sc_seg(x1, src, dst)
  h2 = tc_mid(part3, degp3, W1_1, b1_1.reshape(1, 128))
  part4 = sc_seg(h2, src, dst)
  x2 = tc_res(part4, degp3, W2_1, b2_1.reshape(1, 128), x1)

  return (x2, edge_index)


# speedup vs baseline: 1.1692x; 1.0610x over previous
import functools

import jax
import jax.numpy as jnp
from jax import lax
from jax.experimental import pallas as pl
from jax.experimental.pallas import tpu as pltpu
from jax.experimental.pallas import tpu_sc as plsc

_NC = 2
_NS = 16
_LANES = 128
_F0 = 0.75


@functools.lru_cache(maxsize=None)
def _make_deg_pass(nch, n_pad_deg):
  mesh = plsc.VectorSubcoreMesh(core_axis_name="c", subcore_axis_name="s")
  dpt = n_pad_deg // _NS

  def body(dst_hbm, deg_out, dst_v, ones_v, dzero, deg_sh):
    c = lax.axis_index("c")
    s = lax.axis_index("s")
    w = s * _NC + c

    for k in range(_LANES // 16):
      ones_v[pl.ds(16 * k, 16)] = jnp.ones((16,), jnp.float32)
    for k in range(dpt // 16):
      dzero[pl.ds(16 * k, 16)] = jnp.zeros((16,), jnp.float32)
    pltpu.sync_copy(dzero, deg_sh.at[pl.ds(s * dpt, dpt)])
    plsc.subcore_barrier()

    pltpu.sync_copy(dst_hbm.at[pl.ds(pl.multiple_of(w * nch, 8), nch)], dst_v)

    def step(j, carry):
      pltpu.sync_copy(ones_v, deg_sh.at[dst_v.at[j]], add=True)
      return carry
    lax.fori_loop(0, nch, step, 0)

    plsc.subcore_barrier()
    pltpu.sync_copy(deg_sh.at[pl.ds(s * dpt, dpt)],
                    deg_out.at[pl.ds(c * n_pad_deg + s * dpt, dpt)])

  return pl.kernel(
      body,
      out_type=jax.ShapeDtypeStruct((_NC * n_pad_deg,), jnp.float32),
      mesh=mesh,
      scratch_types=[
          pltpu.VMEM((nch, _LANES), jnp.int32),
          pltpu.VMEM((_LANES,), jnp.float32),
          pltpu.VMEM((dpt,), jnp.float32),
          pltpu.VMEM_SHARED((n_pad_deg,), jnp.float32),
      ],
  )


@functools.lru_cache(maxsize=None)
def _make_sc_pass(nch0, nch1, n_pad):
  mesh = plsc.VectorSubcoreMesh(core_axis_name="c", subcore_axis_name="s")
  rpt = n_pad // _NS
  grp = 8
  tot = nch0 + nch1
  nmax = max(nch0, nch1)

  def body(x_hbm, src_hbm, dst_hbm, part_out, src_v, dst_g, rows0, rows1,
           acc_sh, sem0, sem1, ssem0, ssem1):
    c = lax.axis_index("c")
    s = lax.axis_index("s")
    base = pl.multiple_of(s * tot + c * nch0, 8)

    def zrow(i, carry):
      for k in range(128 // 16):
        rows0[i, pl.ds(16 * k, 16)] = jnp.zeros((16,), jnp.float32)
      return carry
    lax.fori_loop(0, _LANES, zrow, 0)

    off = s * rpt
    nfull = rpt // _LANES
    rem = rpt - nfull * _LANES
    for t in range(nfull):
      pltpu.sync_copy(rows0, acc_sh.at[pl.ds(off + t * _LANES, _LANES)])
    if rem:
      pltpu.sync_copy(rows0.at[pl.ds(0, rem)],
                      acc_sh.at[pl.ds(off + nfull * _LANES, rem)])

    plsc.subcore_barrier()

    pltpu.sync_copy(src_hbm.at[pl.ds(base, nmax)], src_v)
    pltpu.sync_copy(dst_hbm.at[pl.ds(base, grp)], dst_g)
    pltpu.async_copy(x_hbm.at[src_v.at[0]], rows0, sem0)
    pltpu.async_copy(x_hbm.at[src_v.at[1]], rows1, sem1)

    def run(cnt):
      def pair(jj, carry):
        j0 = 2 * jj
        j1 = j0 + 1

        @pl.when(jnp.logical_and(lax.rem(j0, grp) == 0, jj != 0))
        def _():
          g = lax.div(j0, grp)
          pltpu.sync_copy(
              dst_hbm.at[pl.ds(pl.multiple_of(base + g * grp, 8), grp)],
              dst_g)

        r0 = lax.rem(j0, grp)
        pltpu.make_async_copy(x_hbm.at[src_v.at[j0]], rows0, sem0).wait()
        pltpu.async_copy(rows0, acc_sh.at[dst_g.at[r0]], ssem0, add=True)
        pltpu.make_async_copy(x_hbm.at[src_v.at[j1]], rows1, sem1).wait()
        pltpu.async_copy(rows1, acc_sh.at[dst_g.at[r0 + 1]], ssem1,
                         add=True)
        jn0 = jnp.minimum(j0 + 2, cnt - 1)
        jn1 = jnp.minimum(j1 + 2, cnt - 1)
        pltpu.make_async_copy(rows0, acc_sh.at[dst_g.at[r0]], ssem0).wait()
        pltpu.async_copy(x_hbm.at[src_v.at[jn0]], rows0, sem0)
        pltpu.make_async_copy(rows1, acc_sh.at[dst_g.at[r0 + 1]],
                              ssem1).wait()
        pltpu.async_copy(x_hbm.at[src_v.at[jn1]], rows1, sem1)
        return carry
      lax.fori_loop(0, cnt // 2, pair, 0)

      pltpu.make_async_copy(x_hbm.at[src_v.at[cnt - 1]], rows0, sem0).wait()
      pltpu.make_async_copy(x_hbm.at[src_v.at[cnt - 1]], rows1, sem1).wait()

    @pl.when(c == 0)
    def _():
      run(nch0)

    @pl.when(c == 1)
    def _():
      run(nch1)

    plsc.subcore_barrier()

    pltpu.sync_copy(acc_sh.at[pl.ds(s * rpt, rpt)],
                    part_out.at[c, pl.ds(s * rpt, rpt)])

  return pl.kernel(
      body,
      out_type=jax.ShapeDtypeStruct((_NC, n_pad, 128), jnp.float32),
      mesh=mesh,
      scratch_types=[
          pltpu.VMEM((nmax, _LANES), jnp.int32),
          pltpu.VMEM((grp, _LANES), jnp.int32),
          pltpu.VMEM((_LANES, 128), jnp.float32),
          pltpu.VMEM((_LANES, 128), jnp.float32),
          pltpu.VMEM_SHARED((n_pad, 128), jnp.float32),
          pltpu.SemaphoreType.DMA,
          pltpu.SemaphoreType.DMA,
          pltpu.SemaphoreType.DMA,
          pltpu.SemaphoreType.DMA,
      ],
  )


@functools.lru_cache(maxsize=None)
def _make_tc_stage(n, blk, residual):
  grid = (n // blk,)

  def body(part_ref, degp_ref, w_ref, b_ref, *rest):
    if residual:
      x_ref, out_ref = rest
    else:
      (out_ref,) = rest
    p = part_ref[0] + part_ref[1]
    deg = degp_ref[0] + degp_ref[1]
    dinv = 1.0 / jnp.maximum(deg, 1.0)
    h = jnp.dot(p * dinv, w_ref[...],
                preferred_element_type=jnp.float32) + b_ref[...]
    if residual:
      h = h + x_ref[...]
    out_ref[...] = jnp.maximum(h, 0.0)

  in_specs = [
      pl.BlockSpec((_NC, blk, 128), lambda i: (0, i, 0)),
      pl.BlockSpec((_NC, blk, 1), lambda i: (0, i, 0)),
      pl.BlockSpec((128, 128), lambda i: (0, 0)),
      pl.BlockSpec((1, 128), lambda i: (0, 0)),
  ]
  if residual:
    in_specs.append(pl.BlockSpec((blk, 128), lambda i: (i, 0)))

  return pl.pallas_call(
      body,
      grid=grid,
      in_specs=in_specs,
      out_specs=pl.BlockSpec((blk, 128), lambda i: (i, 0)),
      out_shape=jax.ShapeDtypeStruct((n, 128), jnp.float32),
  )


def kernel(x, edge_index, W1_0, b1_0, W2_0, b2_0, W1_1, b1_1, W2_1, b2_1):
  n, d = x.shape
  e = edge_index.shape[1]
  assert d == 128

  workers = _NC * _NS
  tot = 8 * (-(-e // (_NS * _LANES * 8)))
  nch0 = 8 * int(round(tot * _F0 / 8))
  nch0 = min(max(nch0, 8), tot - 8)
  nch1 = tot - nch0
  rows_needed = (_NS - 1) * tot + nch0 + max(nch0, nch1)
  e_pad = max(_NS * tot, rows_needed) * _LANES
  n_pad = 128 * (-(-(n + 1) // 128))
  n_pad_deg = _NS * (16 * (-(-(n_pad // _NS) // 16)))
  nch_deg = tot // 2

  pad = e_pad - e
  src = jnp.concatenate(
      [edge_index[0], jnp.zeros((pad,), jnp.int32)]).reshape(-1, _LANES)
  dst = jnp.concatenate(
      [edge_index[1], jnp.full((pad,), n, jnp.int32)]).reshape(-1, _LANES)

  sc_deg = _make_deg_pass(nch_deg, n_pad_deg)
  sc_seg = _make_sc_pass(nch0, nch1, n_pad)

  blk = 1000 if n % 1000 == 0 else 8 * (n // 8)
  tc_mid = _make_tc_stage(n, blk, False)
  tc_res = _make_tc_stage(n, blk, True)

  degp = sc_deg(dst)
  degp3 = degp.reshape(_NC, n_pad_deg, 1)

  part1 = sc_seg(x, src, dst)
  h = tc_mid(part1, degp3, W1_0, b1_0.reshape(1, 128))
  part2 = sc_seg(h, src, dst)
  x1 = tc_res(part2, degp3, W2_0, b2_0.reshape(1, 128), x)

  part3 = sc_seg(x1, src, dst)
  h2 = tc_mid(part3, degp3, W1_1, b1_1.reshape(1, 128))
  part4 = sc_seg(h2, src, dst)
  x2 = tc_res(part4, degp3, W2_1, b2_1.reshape(1, 128), x1)

  return (x2, edge_index)


# speedup vs baseline: 1.1733x; 1.0035x over previous
import functools

import jax
import jax.numpy as jnp
from jax import lax
from jax.experimental import pallas as pl
from jax.experimental.pallas import tpu as pltpu
from jax.experimental.pallas import tpu_sc as plsc

_NC = 2
_NS = 16
_LANES = 128
_F0 = 0.75


@functools.lru_cache(maxsize=None)
def _make_deg_pass(nch, n_pad_deg):
  mesh = plsc.VectorSubcoreMesh(core_axis_name="c", subcore_axis_name="s")
  dpt = n_pad_deg // _NS

  def body(dst_hbm, deg_out, dst_v, ones_v, dzero, deg_sh):
    c = lax.axis_index("c")
    s = lax.axis_index("s")
    w = s * _NC + c

    for k in range(_LANES // 16):
      ones_v[pl.ds(16 * k, 16)] = jnp.ones((16,), jnp.float32)
    for k in range(dpt // 16):
      dzero[pl.ds(16 * k, 16)] = jnp.zeros((16,), jnp.float32)
    pltpu.sync_copy(dzero, deg_sh.at[pl.ds(s * dpt, dpt)])
    plsc.subcore_barrier()

    pltpu.sync_copy(dst_hbm.at[pl.ds(pl.multiple_of(w * nch, 8), nch)], dst_v)

    def step(j, carry):
      pltpu.sync_copy(ones_v, deg_sh.at[dst_v.at[j]], add=True)
      return carry
    lax.fori_loop(0, nch, step, 0)

    plsc.subcore_barrier()
    pltpu.sync_copy(deg_sh.at[pl.ds(s * dpt, dpt)],
                    deg_out.at[pl.ds(c * n_pad_deg + s * dpt, dpt)])

  return pl.kernel(
      body,
      out_type=jax.ShapeDtypeStruct((_NC * n_pad_deg,), jnp.float32),
      mesh=mesh,
      scratch_types=[
          pltpu.VMEM((nch, _LANES), jnp.int32),
          pltpu.VMEM((_LANES,), jnp.float32),
          pltpu.VMEM((dpt,), jnp.float32),
          pltpu.VMEM_SHARED((n_pad_deg,), jnp.float32),
      ],
  )


@functools.lru_cache(maxsize=None)
def _make_sc_pass(nch0, nch1, n_pad):
  mesh = plsc.VectorSubcoreMesh(core_axis_name="c", subcore_axis_name="s")
  rpt = n_pad // _NS
  grp = 8
  tot = nch0 + nch1
  nmax = max(nch0, nch1)

  def body(x_hbm, src_hbm, dst_hbm, part_out, src_v, dst_g, rows0, rows1,
           acc_sh, sem0, sem1, ssem0, ssem1):
    c = lax.axis_index("c")
    s = lax.axis_index("s")
    base = pl.multiple_of(s * tot + c * nch0, 8)

    def zrow(i, carry):
      for k in range(128 // 16):
        rows0[i, pl.ds(16 * k, 16)] = jnp.zeros((16,), jnp.float32)
      return carry
    lax.fori_loop(0, _LANES, zrow, 0)

    off = s * rpt
    nfull = rpt // _LANES
    rem = rpt - nfull * _LANES
    for t in range(nfull):
      pltpu.sync_copy(rows0, acc_sh.at[pl.ds(off + t * _LANES, _LANES)])
    if rem:
      pltpu.sync_copy(rows0.at[pl.ds(0, rem)],
                      acc_sh.at[pl.ds(off + nfull * _LANES, rem)])

    plsc.subcore_barrier()

    pltpu.sync_copy(src_hbm.at[pl.ds(base, nmax)], src_v)
    pltpu.sync_copy(dst_hbm.at[pl.ds(base, grp)], dst_g)
    pltpu.async_copy(x_hbm.at[src_v.at[0]], rows0, sem0)
    pltpu.async_copy(x_hbm.at[src_v.at[1]], rows1, sem1)

    def run(cnt):
      def pair(jj, carry):
        j0 = 2 * jj
        j1 = j0 + 1

        @pl.when(jnp.logical_and(lax.rem(j0, grp) == 0, jj != 0))
        def _():
          g = lax.div(j0, grp)
          pltpu.sync_copy(
              dst_hbm.at[pl.ds(pl.multiple_of(base + g * grp, 8), grp)],
              dst_g)

        r0 = lax.rem(j0, grp)
        pltpu.make_async_copy(x_hbm.at[src_v.at[j0]], rows0, sem0).wait()
        pltpu.async_copy(rows0, acc_sh.at[dst_g.at[r0]], ssem0, add=True)
        pltpu.make_async_copy(x_hbm.at[src_v.at[j1]], rows1, sem1).wait()
        pltpu.async_copy(rows1, acc_sh.at[dst_g.at[r0 + 1]], ssem1,
                         add=True)
        jn0 = jnp.minimum(j0 + 2, cnt - 1)
        jn1 = jnp.minimum(j1 + 2, cnt - 1)
        pltpu.make_async_copy(rows0, acc_sh.at[dst_g.at[r0]], ssem0).wait()
        pltpu.async_copy(x_hbm.at[src_v.at[jn0]], rows0, sem0)
        pltpu.make_async_copy(rows1, acc_sh.at[dst_g.at[r0 + 1]],
                              ssem1).wait()
        pltpu.async_copy(x_hbm.at[src_v.at[jn1]], rows1, sem1)
        return carry
      lax.fori_loop(0, cnt // 2, pair, 0)

      pltpu.make_async_copy(x_hbm.at[src_v.at[cnt - 1]], rows0, sem0).wait()
      pltpu.make_async_copy(x_hbm.at[src_v.at[cnt - 1]], rows1, sem1).wait()

    @pl.when(c == 0)
    def _():
      run(nch0)

    @pl.when(c == 1)
    def _():
      run(nch1)

    plsc.subcore_barrier()

    pltpu.sync_copy(acc_sh.at[pl.ds(s * rpt, rpt)],
                    part_out.at[c, pl.ds(s * rpt, rpt)])

  return pl.kernel(
      body,
      out_type=jax.ShapeDtypeStruct((_NC, n_pad, 128), jnp.float32),
      mesh=mesh,
      scratch_types=[
          pltpu.VMEM((nmax, _LANES), jnp.int32),
          pltpu.VMEM((grp, _LANES), jnp.int32),
          pltpu.VMEM((_LANES, 128), jnp.float32),
          pltpu.VMEM((_LANES, 128), jnp.float32),
          pltpu.VMEM_SHARED((n_pad, 128), jnp.float32),
          pltpu.SemaphoreType.DMA,
          pltpu.SemaphoreType.DMA,
          pltpu.SemaphoreType.DMA,
          pltpu.SemaphoreType.DMA,
      ],
  )


@functools.lru_cache(maxsize=None)
def _make_tc_stage(n, blk, residual):
  grid = (n // blk,)

  def body(part_ref, degp_ref, w_ref, b_ref, *rest):
    if residual:
      x_ref, out_ref = rest
    else:
      (out_ref,) = rest
    p = part_ref[0] + part_ref[1]
    deg = degp_ref[0] + degp_ref[1]
    dinv = 1.0 / jnp.maximum(deg, 1.0)
    h = jnp.dot(p * dinv, w_ref[...],
                preferred_element_type=jnp.float32) + b_ref[...]
    if residual:
      h = h + x_ref[...]
    out_ref[...] = jnp.maximum(h, 0.0)

  in_specs = [
      pl.BlockSpec((_NC, blk, 128), lambda i: (0, i, 0)),
      pl.BlockSpec((_NC, blk, 1), lambda i: (0, i, 0)),
      pl.BlockSpec((128, 128), lambda i: (0, 0)),
      pl.BlockSpec((1, 128), lambda i: (0, 0)),
  ]
  if residual:
    in_specs.append(pl.BlockSpec((blk, 128), lambda i: (i, 0)))

  return pl.pallas_call(
      body,
      grid=grid,
      in_specs=in_specs,
      out_specs=pl.BlockSpec((blk, 128), lambda i: (i, 0)),
      out_shape=jax.ShapeDtypeStruct((n, 128), jnp.float32),
  )


def kernel(x, edge_index, W1_0, b1_0, W2_0, b2_0, W1_1, b1_1, W2_1, b2_1):
  n, d = x.shape
  e = edge_index.shape[1]
  assert d == 128

  workers = _NC * _NS
  tot = 8 * (-(-e // (_NS * _LANES * 8)))
  nch0 = 8 * int(round(tot * _F0 / 8))
  nch0 = min(max(nch0, 8), tot - 8)
  nch1 = tot - nch0
  rows_needed = (_NS - 1) * tot + nch0 + max(nch0, nch1)
  e_pad = max(_NS * tot, rows_needed) * _LANES
  n_pad = 128 * (-(-(n + 1) // 128))
  n_pad_deg = _NS * (16 * (-(-(n_pad // _NS) // 16)))
  nch_deg = tot // 2

  pad = e_pad - e
  src = jnp.concatenate(
      [edge_index[0], jnp.zeros((pad,), jnp.int32)]).reshape(-1, _LANES)
  dst = jnp.concatenate(
      [edge_index[1], jnp.full((pad,), n, jnp.int32)]).reshape(-1, _LANES)

  sc_deg = _make_deg_pass(nch_deg, n_pad_deg)
  sc_seg = _make_sc_pass(nch0, nch1, n_pad)

  blk = 2000 if n % 2000 == 0 else (1000 if n % 1000 == 0 else 8 * (n // 8))
  tc_mid = _make_tc_stage(n, blk, False)
  tc_res = _make_tc_stage(n, blk, True)

  degp = sc_deg(dst)
  degp3 = degp.reshape(_NC, n_pad_deg, 1)

  part1 = sc_seg(x, src, dst)
  h = tc_mid(part1, degp3, W1_0, b1_0.reshape(1, 128))
  part2 = sc_seg(h, src, dst)
  x1 = tc_res(part2, degp3, W2_0, b2_0.reshape(1, 128), x)

  part3 = sc_seg(x1, src, dst)
  h2 = tc_mid(part3, degp3, W1_1, b1_1.reshape(1, 128))
  part4 = sc_seg(h2, src, dst)
  x2 = tc_res(part4, degp3, W2_1, b2_1.reshape(1, 128), x1)

  return (x2, edge_index)


# speedup vs baseline: 1.1742x; 1.0008x over previous
import functools

import jax
import jax.numpy as jnp
from jax import lax
from jax.experimental import pallas as pl
from jax.experimental.pallas import tpu as pltpu
from jax.experimental.pallas import tpu_sc as plsc

_NC = 2
_NS = 16
_LANES = 128
_F0 = 0.75


@functools.lru_cache(maxsize=None)
def _make_deg_pass(nch, n_pad_deg):
  mesh = plsc.VectorSubcoreMesh(core_axis_name="c", subcore_axis_name="s")
  dpt = n_pad_deg // _NS

  def body(dst_hbm, deg_out, dst_v, ones_v, dzero, deg_sh):
    c = lax.axis_index("c")
    s = lax.axis_index("s")
    w = s * _NC + c

    for k in range(_LANES // 16):
      ones_v[pl.ds(16 * k, 16)] = jnp.ones((16,), jnp.float32)
    for k in range(dpt // 16):
      dzero[pl.ds(16 * k, 16)] = jnp.zeros((16,), jnp.float32)
    pltpu.sync_copy(dzero, deg_sh.at[pl.ds(s * dpt, dpt)])
    plsc.subcore_barrier()

    pltpu.sync_copy(dst_hbm.at[pl.ds(pl.multiple_of(w * nch, 8), nch)], dst_v)

    def step(j, carry):
      pltpu.sync_copy(ones_v, deg_sh.at[dst_v.at[j]], add=True)
      return carry
    lax.fori_loop(0, nch, step, 0)

    plsc.subcore_barrier()
    pltpu.sync_copy(deg_sh.at[pl.ds(s * dpt, dpt)],
                    deg_out.at[pl.ds(c * n_pad_deg + s * dpt, dpt)])

  return pl.kernel(
      body,
      out_type=jax.ShapeDtypeStruct((_NC * n_pad_deg,), jnp.float32),
      mesh=mesh,
      scratch_types=[
          pltpu.VMEM((nch, _LANES), jnp.int32),
          pltpu.VMEM((_LANES,), jnp.float32),
          pltpu.VMEM((dpt,), jnp.float32),
          pltpu.VMEM_SHARED((n_pad_deg,), jnp.float32),
      ],
  )


@functools.lru_cache(maxsize=None)
def _make_sc_pass(nch0, nch1, n_pad):
  mesh = plsc.VectorSubcoreMesh(core_axis_name="c", subcore_axis_name="s")
  rpt = n_pad // _NS
  grp = 8
  tot = nch0 + nch1
  nmax = max(nch0, nch1)

  def body(x_hbm, src_hbm, dst_hbm, part_out, src_v, dst_g, rows0, rows1,
           acc_sh, sem0, sem1, ssem0, ssem1):
    c = lax.axis_index("c")
    s = lax.axis_index("s")
    base = pl.multiple_of(s * tot + c * nch0, 8)

    def zrow(i, carry):
      for k in range(128 // 16):
        rows0[i, pl.ds(16 * k, 16)] = jnp.zeros((16,), jnp.float32)
      return carry
    lax.fori_loop(0, _LANES, zrow, 0)

    off = s * rpt
    nfull = rpt // _LANES
    rem = rpt - nfull * _LANES
    for t in range(nfull):
      pltpu.sync_copy(rows0, acc_sh.at[pl.ds(off + t * _LANES, _LANES)])
    if rem:
      pltpu.sync_copy(rows0.at[pl.ds(0, rem)],
                      acc_sh.at[pl.ds(off + nfull * _LANES, rem)])

    plsc.subcore_barrier()

    pltpu.sync_copy(src_hbm.at[pl.ds(base, nmax)], src_v)
    pltpu.sync_copy(dst_hbm.at[pl.ds(base, grp)], dst_g)
    pltpu.async_copy(x_hbm.at[src_v.at[0]], rows0, sem0)
    pltpu.async_copy(x_hbm.at[src_v.at[1]], rows1, sem1)

    def run(cnt):
      def pair(jj, carry):
        j0 = 2 * jj
        j1 = j0 + 1

        @pl.when(jnp.logical_and(lax.rem(j0, grp) == 0, jj != 0))
        def _():
          g = lax.div(j0, grp)
          pltpu.sync_copy(
              dst_hbm.at[pl.ds(pl.multiple_of(base + g * grp, 8), grp)],
              dst_g)

        r0 = lax.rem(j0, grp)
        pltpu.make_async_copy(x_hbm.at[src_v.at[j0]], rows0, sem0).wait()
        pltpu.async_copy(rows0, acc_sh.at[dst_g.at[r0]], ssem0, add=True)
        pltpu.make_async_copy(x_hbm.at[src_v.at[j1]], rows1, sem1).wait()
        pltpu.async_copy(rows1, acc_sh.at[dst_g.at[r0 + 1]], ssem1,
                         add=True)
        jn0 = jnp.minimum(j0 + 2, cnt - 1)
        jn1 = jnp.minimum(j1 + 2, cnt - 1)
        pltpu.make_async_copy(rows0, acc_sh.at[dst_g.at[r0]], ssem0).wait()
        pltpu.async_copy(x_hbm.at[src_v.at[jn0]], rows0, sem0)
        pltpu.make_async_copy(rows1, acc_sh.at[dst_g.at[r0 + 1]],
                              ssem1).wait()
        pltpu.async_copy(x_hbm.at[src_v.at[jn1]], rows1, sem1)
        return carry
      lax.fori_loop(0, cnt // 2, pair, 0)

      pltpu.make_async_copy(x_hbm.at[src_v.at[cnt - 1]], rows0, sem0).wait()
      pltpu.make_async_copy(x_hbm.at[src_v.at[cnt - 1]], rows1, sem1).wait()

    @pl.when(c == 0)
    def _():
      run(nch0)

    @pl.when(c == 1)
    def _():
      run(nch1)

    plsc.subcore_barrier()

    pltpu.sync_copy(acc_sh.at[pl.ds(s * rpt, rpt)],
                    part_out.at[c, pl.ds(s * rpt, rpt)])

  return pl.kernel(
      body,
      out_type=jax.ShapeDtypeStruct((_NC, n_pad, 128), jnp.float32),
      mesh=mesh,
      scratch_types=[
          pltpu.VMEM((nmax, _LANES), jnp.int32),
          pltpu.VMEM((grp, _LANES), jnp.int32),
          pltpu.VMEM((_LANES, 128), jnp.float32),
          pltpu.VMEM((_LANES, 128), jnp.float32),
          pltpu.VMEM_SHARED((n_pad, 128), jnp.float32),
          pltpu.SemaphoreType.DMA,
          pltpu.SemaphoreType.DMA,
          pltpu.SemaphoreType.DMA,
          pltpu.SemaphoreType.DMA,
      ],
  )


@functools.lru_cache(maxsize=None)
def _make_tc_stage(n, blk, residual):
  grid = (n // blk,)

  def body(part_ref, degp_ref, w_ref, b_ref, *rest):
    if residual:
      x_ref, out_ref = rest
    else:
      (out_ref,) = rest
    p = part_ref[0] + part_ref[1]
    deg = degp_ref[0] + degp_ref[1]
    dinv = 1.0 / jnp.maximum(deg, 1.0)
    h = jnp.dot(p * dinv, w_ref[...],
                preferred_element_type=jnp.float32) + b_ref[...]
    if residual:
      h = h + x_ref[...]
    out_ref[...] = jnp.maximum(h, 0.0)

  in_specs = [
      pl.BlockSpec((_NC, blk, 128), lambda i: (0, i, 0)),
      pl.BlockSpec((_NC, blk, 1), lambda i: (0, i, 0)),
      pl.BlockSpec((128, 128), lambda i: (0, 0)),
      pl.BlockSpec((1, 128), lambda i: (0, 0)),
  ]
  if residual:
    in_specs.append(pl.BlockSpec((blk, 128), lambda i: (i, 0)))

  return pl.pallas_call(
      body,
      grid=grid,
      in_specs=in_specs,
      out_specs=pl.BlockSpec((blk, 128), lambda i: (i, 0)),
      out_shape=jax.ShapeDtypeStruct((n, 128), jnp.float32),
  )


def kernel(x, edge_index, W1_0, b1_0, W2_0, b2_0, W1_1, b1_1, W2_1, b2_1):
  n, d = x.shape
  e = edge_index.shape[1]
  assert d == 128

  tot = 16 * (-(-e // (_NS * _LANES * 16)))
  nch0 = 8 * int(round(tot * _F0 / 8))
  nch0 = min(max(nch0, 8), tot - 8)
  nch1 = tot - nch0
  rows_needed = (_NS - 1) * tot + nch0 + max(nch0, nch1)
  e_pad = max(_NS * tot, rows_needed) * _LANES
  n_pad = 128 * (-(-(n + 1) // 128))
  n_pad_deg = _NS * (16 * (-(-(n_pad // _NS) // 16)))
  nch_deg = tot // 2

  pad = e_pad - e
  src = jnp.concatenate(
      [edge_index[0], jnp.zeros((pad,), jnp.int32)]).reshape(-1, _LANES)
  dst = jnp.concatenate(
      [edge_index[1], jnp.full((pad,), n, jnp.int32)]).reshape(-1, _LANES)

  sc_deg = _make_deg_pass(nch_deg, n_pad_deg)
  sc_seg = _make_sc_pass(nch0, nch1, n_pad)

  blk = 2000 if n % 2000 == 0 else (1000 if n % 1000 == 0 else 8 * (n // 8))
  tc_mid = _make_tc_stage(n, blk, False)
  tc_res = _make_tc_stage(n, blk, True)

  degp = sc_deg(dst)
  degp3 = degp.reshape(_NC, n_pad_deg, 1)

  part1 = sc_seg(x, src, dst)
  h = tc_mid(part1, degp3, W1_0, b1_0.reshape(1, 128))
  part2 = sc_seg(h, src, dst)
  x1 = tc_res(part2, degp3, W2_0, b2_0.reshape(1, 128), x)

  part3 = sc_seg(x1, src, dst)
  h2 = tc_mid(part3, degp3, W1_1, b1_1.reshape(1, 128))
  part4 = sc_seg(h2, src, dst)
  x2 = tc_res(part4, degp3, W2_1, b2_1.reshape(1, 128), x1)

  return (x2, edge_index)
